# baseline (speedup 1.0000x reference)
import functools

import jax
import jax.numpy as jnp
from jax import lax
from jax.experimental import pallas as pl
from jax.experimental.pallas import tpu as pltpu
from jax.experimental.pallas import tpu_sc as plsc

N = 10000
D = 128
G = 64
OUT = 10
NC = 2
NS = 16
NW = NC * NS
CHUNK = 96
DBLK = 16
RING = 2
N_PAD = 10240
ZROWS = 16

_mesh = plsc.VectorSubcoreMesh(
    core_axis_name="c", subcore_axis_name="s", num_cores=NC, num_subcores=NS)


def _worker():
  c = lax.axis_index("c")
  s = lax.axis_index("s")
  return c, s, c * NS + s


def _deg_body(cpt, didx_hbm, out_hbm, didx_v, ones_v, deg_sh):
  c, s, w = _worker()
  zslice = N_PAD // NS
  zc = 32
  for q in range(CHUNK // 16):
    ones_v[pl.ds(q * 16, 16)] = jnp.zeros((16,), jnp.float32)
  for k in range(zslice // zc):
    pltpu.sync_copy(ones_v.at[pl.ds(0, zc)],
                    deg_sh.at[pl.ds(s * zslice + k * zc, zc)])
  for q in range(CHUNK // 16):
    ones_v[pl.ds(q * 16, 16)] = jnp.ones((16,), jnp.float32)
  plsc.subcore_barrier()

  def blk(b, carry):
    pltpu.sync_copy(didx_hbm.at[pl.ds(w * cpt + b * DBLK, DBLK)], didx_v)

    def body(j, carry2):
      pltpu.sync_copy(ones_v, deg_sh.at[didx_v.at[j]], add=True)
      return carry2

    return lax.fori_loop(0, DBLK, body, carry)

  lax.fori_loop(0, cpt // DBLK, blk, 0)
  plsc.subcore_barrier()
  pltpu.sync_copy(deg_sh.at[pl.ds(s * zslice, zslice)],
                  out_hbm.at[pl.ds(c * N_PAD + s * zslice, zslice)])


def _scat_body(cpt, g_hbm, sidx_hbm, didx_hbm, out_hbm,
               sidx_v, didx_v, rows_0, rows_1, acc_sh,
               gs0, gs1, ss0, ss1):
  c, s, w = _worker()
  bufs = (rows_0, rows_1)
  gsem = (gs0, gs1)
  ssem = (ss0, ss1)
  for r in range(ZROWS):
    for q in range(D // 16):
      rows_0[r, pl.ds(q * 16, 16)] = jnp.zeros((16,), jnp.float32)
  zslice = N_PAD // NS

  def zbody(k, carry):
    pltpu.sync_copy(rows_0.at[pl.ds(0, ZROWS)],
                    acc_sh.at[pl.ds(s * zslice + k * ZROWS, ZROWS)])
    return carry

  lax.fori_loop(0, zslice // ZROWS, zbody, 0)
  plsc.subcore_barrier()

  def block(b, carry):
    pltpu.sync_copy(sidx_hbm.at[pl.ds(w * cpt + b * DBLK, DBLK)], sidx_v)
    pltpu.sync_copy(didx_hbm.at[pl.ds(w * cpt + b * DBLK, DBLK)], didx_v)
    for l in range(RING - 1):
      pltpu.async_copy(g_hbm.at[sidx_v.at[l]], bufs[l], gsem[l])
    for l in range(DBLK):
      r = l % RING
      pltpu.make_async_copy(g_hbm.at[sidx_v.at[l]], bufs[r], gsem[r]).wait()
      pltpu.async_copy(bufs[r], acc_sh.at[didx_v.at[l]], ssem[r], add=True)
      nl = l + RING - 1
      if nl < DBLK:
        rn = nl % RING
        if nl >= RING:
          pltpu.make_async_copy(bufs[rn], acc_sh.at[didx_v.at[nl - RING]],
                                ssem[rn]).wait()
        pltpu.async_copy(g_hbm.at[sidx_v.at[nl]], bufs[rn], gsem[rn])
    for l in range(DBLK - RING, DBLK):
      r = l % RING
      pltpu.make_async_copy(bufs[r], acc_sh.at[didx_v.at[l]], ssem[r]).wait()
    return carry

  lax.fori_loop(0, cpt // DBLK, block, 0)
  plsc.subcore_barrier()
  pltpu.sync_copy(acc_sh.at[pl.ds(s * zslice, zslice)],
                  out_hbm.at[pl.ds(c * N_PAD + s * zslice, zslice)])


def _make_sc_kernels(cpt):
  deg = pl.kernel(
      functools.partial(_deg_body, cpt),
      out_type=jax.ShapeDtypeStruct((NC * N_PAD,), jnp.float32),
      mesh=_mesh,
      scratch_types=[
          pltpu.VMEM((DBLK, CHUNK), jnp.int32),
          pltpu.VMEM((CHUNK,), jnp.float32),
          pltpu.VMEM_SHARED((N_PAD,), jnp.float32),
      ],
  )
  scat = pl.kernel(
      functools.partial(_scat_body, cpt),
      out_type=jax.ShapeDtypeStruct((NC * N_PAD, D), jnp.float32),
      mesh=_mesh,
      scratch_types=[
          pltpu.VMEM((DBLK, CHUNK), jnp.int32),
          pltpu.VMEM((DBLK, CHUNK), jnp.int32),
          pltpu.VMEM((CHUNK, D), jnp.float32),
          pltpu.VMEM((CHUNK, D), jnp.float32),
          pltpu.VMEM_SHARED((N_PAD, D), jnp.float32),
          pltpu.SemaphoreType.DMA,
          pltpu.SemaphoreType.DMA,
          pltpu.SemaphoreType.DMA,
          pltpu.SemaphoreType.DMA,
      ],
  )
  return deg, scat


def _pre_body(dp_ref, x_ref, w_ref, g_ref, dinv_ref):
  deg = dp_ref[:, 0:1] + dp_ref[:, 1:2] + 1.0
  dinv = lax.rsqrt(deg)
  dinv_ref[...] = dinv
  g_ref[...] = dinv * jnp.dot(x_ref[...], w_ref[...],
                              preferred_element_type=jnp.float32)


def _mid_body(p_ref, g_ref, dinv_ref, b_ref, a_ref, w_ref, go_ref):
  dinv = dinv_ref[...]
  h = dinv * (p_ref[0:N] + p_ref[N_PAD:N_PAD + N] + g_ref[...]) + b_ref[...]
  a = a_ref[0, 0]
  x2 = jnp.where(h > 0, h, a * h)
  go_ref[...] = dinv * jnp.dot(x2, w_ref[...],
                               preferred_element_type=jnp.float32)


def _post_body(p_ref, g_ref, dinv_ref, b_ref, batch_ref, lw_ref, lb_ref,
               out_ref):
  h = dinv_ref[...] * (p_ref[0:N] + p_ref[N_PAD:N_PAD + N] + g_ref[...]) \
      + b_ref[...]
  y = jnp.dot(h, lw_ref[...], preferred_element_type=jnp.float32)
  gid = lax.broadcasted_iota(jnp.int32, (G, 1), 0)
  oht = (gid == batch_ref[...]).astype(jnp.float32)
  seg = jnp.dot(oht, y, preferred_element_type=jnp.float32)
  cnt = jnp.sum(oht, axis=1, keepdims=True)
  logits = seg / jnp.maximum(cnt, 1.0) + lb_ref[...]
  m = jnp.max(logits, axis=1, keepdims=True)
  z = logits - m
  out_ref[...] = z - jnp.log(jnp.sum(jnp.exp(z), axis=1, keepdims=True))


_pre = pl.pallas_call(
    _pre_body,
    out_shape=(jax.ShapeDtypeStruct((N, D), jnp.float32),
               jax.ShapeDtypeStruct((N, 1), jnp.float32)))

_mid = pl.pallas_call(
    _mid_body, out_shape=jax.ShapeDtypeStruct((N, D), jnp.float32))

_post = pl.pallas_call(
    _post_body, out_shape=jax.ShapeDtypeStruct((G, OUT), jnp.float32))


def kernel(x, edge_index, batch, W0, b0, W1, b1, W2, b2, a0, a1, lin_W, lin_b):
  e = edge_index.shape[1]
  cpt = -(-e // (NW * CHUNK))
  cpt = -(-cpt // DBLK) * DBLK
  e_pad = NW * CHUNK * cpt
  pad = e_pad - e
  pid = jnp.arange(pad, dtype=jnp.int32)
  src = jnp.concatenate([edge_index[0], pid % N]).reshape(NW * cpt, CHUNK)
  dst = jnp.concatenate([edge_index[1], N + pid % (N_PAD - N)]
                        ).reshape(NW * cpt, CHUNK)

  deg_fn, scat_fn = _make_sc_kernels(cpt)

  degp = deg_fn(dst)
  dp = degp.reshape(NC, N_PAD)[:, :N].T

  g0, dinv = _pre(dp, x, W0)
  w_st = jnp.stack([W1, W2, W2])
  b_st = jnp.stack([b0.reshape(1, D), b1.reshape(1, D), b2.reshape(1, D)])
  a_st = jnp.stack([a0.reshape(1, 1), a1.reshape(1, 1), a1.reshape(1, 1)])
  pz = jnp.zeros((NC * N_PAD, D), jnp.float32)

  def layer(i, carry):
    g, _, _ = carry
    p = scat_fn(g, src, dst)
    go = _mid(p, g, dinv,
              lax.dynamic_index_in_dim(b_st, i, keepdims=False),
              lax.dynamic_index_in_dim(a_st, i, keepdims=False),
              lax.dynamic_index_in_dim(w_st, i, keepdims=False))
    return (go, g, p)

  trips = 3 + batch[0] // jnp.int32(2**31 - 1)
  _, g, p = lax.fori_loop(0, trips, layer, (g0, g0, pz))
  return _post(p, g, dinv, b2.reshape(1, D),
               batch.reshape(1, N), lin_W, lin_b.reshape(1, OUT))


# speedup vs baseline: 1.2146x; 1.2146x over previous
import functools

import jax
import jax.numpy as jnp
from jax import lax
from jax.experimental import pallas as pl
from jax.experimental.pallas import tpu as pltpu
from jax.experimental.pallas import tpu_sc as plsc

N = 10000
D = 128
G = 64
OUT = 10
NC = 2
NS = 16
NW = NC * NS
CHUNK = 64
DBLK = 24
RING = 4
N_PAD = 10240
ZROWS = 16

_mesh = plsc.VectorSubcoreMesh(
    core_axis_name="c", subcore_axis_name="s", num_cores=NC, num_subcores=NS)


def _worker():
  c = lax.axis_index("c")
  s = lax.axis_index("s")
  return c, s, c * NS + s


def _deg_body(cpt, didx_hbm, out_hbm, didx_v, ones_v, deg_sh):
  c, s, w = _worker()
  zslice = N_PAD // NS
  zc = 32
  for q in range(CHUNK // 16):
    ones_v[pl.ds(q * 16, 16)] = jnp.zeros((16,), jnp.float32)
  for k in range(zslice // zc):
    pltpu.sync_copy(ones_v.at[pl.ds(0, zc)],
                    deg_sh.at[pl.ds(s * zslice + k * zc, zc)])
  for q in range(CHUNK // 16):
    ones_v[pl.ds(q * 16, 16)] = jnp.ones((16,), jnp.float32)
  plsc.subcore_barrier()

  def blk(b, carry):
    pltpu.sync_copy(didx_hbm.at[pl.ds(w * cpt + b * DBLK, DBLK)], didx_v)

    def body(j, carry2):
      pltpu.sync_copy(ones_v, deg_sh.at[didx_v.at[j]], add=True)
      return carry2

    return lax.fori_loop(0, DBLK, body, carry)

  lax.fori_loop(0, cpt // DBLK, blk, 0)
  plsc.subcore_barrier()
  pltpu.sync_copy(deg_sh.at[pl.ds(s * zslice, zslice)],
                  out_hbm.at[pl.ds(c * N_PAD + s * zslice, zslice)])


def _scat_body(cpt, g_hbm, sidx_hbm, didx_hbm, out_hbm,
               sidx_v, didx_v, rows_0, rows_1, rows_2, rows_3, acc_sh,
               gs0, gs1, gs2, gs3, ss0, ss1, ss2, ss3):
  c, s, w = _worker()
  bufs = (rows_0, rows_1, rows_2, rows_3)
  gsem = (gs0, gs1, gs2, gs3)
  ssem = (ss0, ss1, ss2, ss3)
  for r in range(ZROWS):
    for q in range(D // 16):
      rows_0[r, pl.ds(q * 16, 16)] = jnp.zeros((16,), jnp.float32)
  zslice = N_PAD // NS

  def zbody(k, carry):
    pltpu.sync_copy(rows_0.at[pl.ds(0, ZROWS)],
                    acc_sh.at[pl.ds(s * zslice + k * ZROWS, ZROWS)])
    return carry

  lax.fori_loop(0, zslice // ZROWS, zbody, 0)
  plsc.subcore_barrier()

  def block(b, carry):
    pltpu.sync_copy(sidx_hbm.at[pl.ds(w * cpt + b * DBLK, DBLK)], sidx_v)
    pltpu.sync_copy(didx_hbm.at[pl.ds(w * cpt + b * DBLK, DBLK)], didx_v)
    for l in range(RING - 1):
      pltpu.async_copy(g_hbm.at[sidx_v.at[l]], bufs[l], gsem[l])
    for l in range(DBLK):
      r = l % RING
      pltpu.make_async_copy(g_hbm.at[sidx_v.at[l]], bufs[r], gsem[r]).wait()
      pltpu.async_copy(bufs[r], acc_sh.at[didx_v.at[l]], ssem[r], add=True)
      nl = l + RING - 1
      if nl < DBLK:
        rn = nl % RING
        if nl >= RING:
          pltpu.make_async_copy(bufs[rn], acc_sh.at[didx_v.at[nl - RING]],
                                ssem[rn]).wait()
        pltpu.async_copy(g_hbm.at[sidx_v.at[nl]], bufs[rn], gsem[rn])
    for l in range(DBLK - RING, DBLK):
      r = l % RING
      pltpu.make_async_copy(bufs[r], acc_sh.at[didx_v.at[l]], ssem[r]).wait()
    return carry

  lax.fori_loop(0, cpt // DBLK, block, 0)
  plsc.subcore_barrier()
  pltpu.sync_copy(acc_sh.at[pl.ds(s * zslice, zslice)],
                  out_hbm.at[pl.ds(c * N_PAD + s * zslice, zslice)])


def _make_sc_kernels(cpt):
  deg = pl.kernel(
      functools.partial(_deg_body, cpt),
      out_type=jax.ShapeDtypeStruct((NC * N_PAD,), jnp.float32),
      mesh=_mesh,
      scratch_types=[
          pltpu.VMEM((DBLK, CHUNK), jnp.int32),
          pltpu.VMEM((CHUNK,), jnp.float32),
          pltpu.VMEM_SHARED((N_PAD,), jnp.float32),
      ],
  )
  scat = pl.kernel(
      functools.partial(_scat_body, cpt),
      out_type=jax.ShapeDtypeStruct((NC * N_PAD, D), jnp.float32),
      mesh=_mesh,
      scratch_types=[
          pltpu.VMEM((DBLK, CHUNK), jnp.int32),
          pltpu.VMEM((DBLK, CHUNK), jnp.int32),
          pltpu.VMEM((CHUNK, D), jnp.float32),
          pltpu.VMEM((CHUNK, D), jnp.float32),
          pltpu.VMEM((CHUNK, D), jnp.float32),
          pltpu.VMEM((CHUNK, D), jnp.float32),
          pltpu.VMEM_SHARED((N_PAD, D), jnp.float32),
          pltpu.SemaphoreType.DMA,
          pltpu.SemaphoreType.DMA,
          pltpu.SemaphoreType.DMA,
          pltpu.SemaphoreType.DMA,
          pltpu.SemaphoreType.DMA,
          pltpu.SemaphoreType.DMA,
          pltpu.SemaphoreType.DMA,
          pltpu.SemaphoreType.DMA,
      ],
  )
  return deg, scat


def _pre_body(dp_ref, x_ref, w_ref, g_ref, dinv_ref):
  deg = dp_ref[:, 0:1] + dp_ref[:, 1:2] + 1.0
  dinv = lax.rsqrt(deg)
  dinv_ref[...] = dinv
  g_ref[...] = dinv * jnp.dot(x_ref[...], w_ref[...],
                              preferred_element_type=jnp.float32)


def _mid_body(p_ref, g_ref, dinv_ref, b_ref, a_ref, w_ref, go_ref):
  dinv = dinv_ref[...]
  h = dinv * (p_ref[0:N] + p_ref[N_PAD:N_PAD + N] + g_ref[...]) + b_ref[...]
  a = a_ref[0, 0]
  x2 = jnp.where(h > 0, h, a * h)
  go_ref[...] = dinv * jnp.dot(x2, w_ref[...],
                               preferred_element_type=jnp.float32)


def _post_body(p_ref, g_ref, dinv_ref, b_ref, batch_ref, lw_ref, lb_ref,
               out_ref):
  h = dinv_ref[...] * (p_ref[0:N] + p_ref[N_PAD:N_PAD + N] + g_ref[...]) \
      + b_ref[...]
  y = jnp.dot(h, lw_ref[...], preferred_element_type=jnp.float32)
  gid = lax.broadcasted_iota(jnp.int32, (G, 1), 0)
  oht = (gid == batch_ref[...]).astype(jnp.float32)
  seg = jnp.dot(oht, y, preferred_element_type=jnp.float32)
  cnt = jnp.sum(oht, axis=1, keepdims=True)
  logits = seg / jnp.maximum(cnt, 1.0) + lb_ref[...]
  m = jnp.max(logits, axis=1, keepdims=True)
  z = logits - m
  out_ref[...] = z - jnp.log(jnp.sum(jnp.exp(z), axis=1, keepdims=True))


_pre = pl.pallas_call(
    _pre_body,
    out_shape=(jax.ShapeDtypeStruct((N, D), jnp.float32),
               jax.ShapeDtypeStruct((N, 1), jnp.float32)))

_mid = pl.pallas_call(
    _mid_body, out_shape=jax.ShapeDtypeStruct((N, D), jnp.float32))

_post = pl.pallas_call(
    _post_body, out_shape=jax.ShapeDtypeStruct((G, OUT), jnp.float32))


def kernel(x, edge_index, batch, W0, b0, W1, b1, W2, b2, a0, a1, lin_W, lin_b):
  e = edge_index.shape[1]
  cpt = -(-e // (NW * CHUNK))
  cpt = -(-cpt // DBLK) * DBLK
  e_pad = NW * CHUNK * cpt
  pad = e_pad - e
  pid = jnp.arange(pad, dtype=jnp.int32)
  src = jnp.concatenate([edge_index[0], pid % N]).reshape(NW * cpt, CHUNK)
  dst = jnp.concatenate([edge_index[1], N + pid % (N_PAD - N)]
                        ).reshape(NW * cpt, CHUNK)

  deg_fn, scat_fn = _make_sc_kernels(cpt)

  degp = deg_fn(dst)
  dp = degp.reshape(NC, N_PAD)[:, :N].T

  g0, dinv = _pre(dp, x, W0)
  w_st = jnp.stack([W1, W2, W2])
  b_st = jnp.stack([b0.reshape(1, D), b1.reshape(1, D), b2.reshape(1, D)])
  a_st = jnp.stack([a0.reshape(1, 1), a1.reshape(1, 1), a1.reshape(1, 1)])
  pz = jnp.zeros((NC * N_PAD, D), jnp.float32)

  def layer(i, carry):
    g, _, _ = carry
    p = scat_fn(g, src, dst)
    go = _mid(p, g, dinv,
              lax.dynamic_index_in_dim(b_st, i, keepdims=False),
              lax.dynamic_index_in_dim(a_st, i, keepdims=False),
              lax.dynamic_index_in_dim(w_st, i, keepdims=False))
    return (go, g, p)

  trips = 3 + batch[0] // jnp.int32(2**31 - 1)
  _, g, p = lax.fori_loop(0, trips, layer, (g0, g0, pz))
  return _post(p, g, dinv, b2.reshape(1, D),
               batch.reshape(1, N), lin_W, lin_b.reshape(1, OUT))


# speedup vs baseline: 1.2546x; 1.0329x over previous
import functools

import jax
import jax.numpy as jnp
from jax import lax
from jax.experimental import pallas as pl
from jax.experimental.pallas import tpu as pltpu
from jax.experimental.pallas import tpu_sc as plsc

N = 10000
D = 128
G = 64
OUT = 10
NC = 2
NS = 16
NW = NC * NS
CHUNK = 96
DBLK = 16
RING = 3
N_PAD = 10240
ZROWS = 16

_mesh = plsc.VectorSubcoreMesh(
    core_axis_name="c", subcore_axis_name="s", num_cores=NC, num_subcores=NS)


def _worker():
  c = lax.axis_index("c")
  s = lax.axis_index("s")
  return c, s, c * NS + s


def _deg_body(cpt, didx_hbm, out_hbm, didx_v, ones_v, deg_sh):
  c, s, w = _worker()
  zslice = N_PAD // NS
  zc = 32
  for q in range(CHUNK // 16):
    ones_v[pl.ds(q * 16, 16)] = jnp.zeros((16,), jnp.float32)
  for k in range(zslice // zc):
    pltpu.sync_copy(ones_v.at[pl.ds(0, zc)],
                    deg_sh.at[pl.ds(s * zslice + k * zc, zc)])
  for q in range(CHUNK // 16):
    ones_v[pl.ds(q * 16, 16)] = jnp.ones((16,), jnp.float32)
  plsc.subcore_barrier()

  def blk(b, carry):
    pltpu.sync_copy(didx_hbm.at[pl.ds(w * cpt + b * DBLK, DBLK)], didx_v)

    def body(j, carry2):
      pltpu.sync_copy(ones_v, deg_sh.at[didx_v.at[j]], add=True)
      return carry2

    return lax.fori_loop(0, DBLK, body, carry)

  lax.fori_loop(0, cpt // DBLK, blk, 0)
  plsc.subcore_barrier()
  pltpu.sync_copy(deg_sh.at[pl.ds(s * zslice, zslice)],
                  out_hbm.at[pl.ds(c * N_PAD + s * zslice, zslice)])


def _scat_body(cpt, g_hbm, sidx_hbm, didx_hbm, out_hbm,
               sidx_v, didx_v, rows_0, rows_1, rows_2, acc_sh,
               gs0, gs1, gs2, ss0, ss1, ss2):
  c, s, w = _worker()
  bufs = (rows_0, rows_1, rows_2)
  gsem = (gs0, gs1, gs2)
  ssem = (ss0, ss1, ss2)
  for r in range(ZROWS):
    for q in range(D // 16):
      rows_0[r, pl.ds(q * 16, 16)] = jnp.zeros((16,), jnp.float32)
  zslice = N_PAD // NS

  def zbody(k, carry):
    pltpu.sync_copy(rows_0.at[pl.ds(0, ZROWS)],
                    acc_sh.at[pl.ds(s * zslice + k * ZROWS, ZROWS)])
    return carry

  lax.fori_loop(0, zslice // ZROWS, zbody, 0)
  plsc.subcore_barrier()

  def block(b, carry):
    pltpu.sync_copy(sidx_hbm.at[pl.ds(w * cpt + b * DBLK, DBLK)], sidx_v)
    pltpu.sync_copy(didx_hbm.at[pl.ds(w * cpt + b * DBLK, DBLK)], didx_v)
    for l in range(RING - 1):
      pltpu.async_copy(g_hbm.at[sidx_v.at[l]], bufs[l], gsem[l])
    for l in range(DBLK):
      r = l % RING
      pltpu.make_async_copy(g_hbm.at[sidx_v.at[l]], bufs[r], gsem[r]).wait()
      pltpu.async_copy(bufs[r], acc_sh.at[didx_v.at[l]], ssem[r], add=True)
      nl = l + RING - 1
      if nl < DBLK:
        rn = nl % RING
        if nl >= RING:
          pltpu.make_async_copy(bufs[rn], acc_sh.at[didx_v.at[nl - RING]],
                                ssem[rn]).wait()
        pltpu.async_copy(g_hbm.at[sidx_v.at[nl]], bufs[rn], gsem[rn])
    for l in range(DBLK - RING, DBLK):
      r = l % RING
      pltpu.make_async_copy(bufs[r], acc_sh.at[didx_v.at[l]], ssem[r]).wait()
    return carry

  lax.fori_loop(0, cpt // DBLK, block, 0)
  plsc.subcore_barrier()
  pltpu.sync_copy(acc_sh.at[pl.ds(s * zslice, zslice)],
                  out_hbm.at[pl.ds(c * N_PAD + s * zslice, zslice)])


def _make_sc_kernels(cpt):
  deg = pl.kernel(
      functools.partial(_deg_body, cpt),
      out_type=jax.ShapeDtypeStruct((NC * N_PAD,), jnp.float32),
      mesh=_mesh,
      scratch_types=[
          pltpu.VMEM((DBLK, CHUNK), jnp.int32),
          pltpu.VMEM((CHUNK,), jnp.float32),
          pltpu.VMEM_SHARED((N_PAD,), jnp.float32),
      ],
  )
  scat = pl.kernel(
      functools.partial(_scat_body, cpt),
      out_type=jax.ShapeDtypeStruct((NC * N_PAD, D), jnp.float32),
      mesh=_mesh,
      scratch_types=[
          pltpu.VMEM((DBLK, CHUNK), jnp.int32),
          pltpu.VMEM((DBLK, CHUNK), jnp.int32),
          pltpu.VMEM((CHUNK, D), jnp.float32),
          pltpu.VMEM((CHUNK, D), jnp.float32),
          pltpu.VMEM((CHUNK, D), jnp.float32),
          pltpu.VMEM_SHARED((N_PAD, D), jnp.float32),
          pltpu.SemaphoreType.DMA,
          pltpu.SemaphoreType.DMA,
          pltpu.SemaphoreType.DMA,
          pltpu.SemaphoreType.DMA,
          pltpu.SemaphoreType.DMA,
          pltpu.SemaphoreType.DMA,
      ],
  )
  return deg, scat


def _pre_body(dp_ref, x_ref, w_ref, g_ref, dinv_ref):
  deg = dp_ref[:, 0:1] + dp_ref[:, 1:2] + 1.0
  dinv = lax.rsqrt(deg)
  dinv_ref[...] = dinv
  g_ref[...] = dinv * jnp.dot(x_ref[...], w_ref[...],
                              preferred_element_type=jnp.float32)


def _mid_body(p_ref, g_ref, dinv_ref, b_ref, a_ref, w_ref, go_ref):
  dinv = dinv_ref[...]
  h = dinv * (p_ref[0:N] + p_ref[N_PAD:N_PAD + N] + g_ref[...]) + b_ref[...]
  a = a_ref[0, 0]
  x2 = jnp.where(h > 0, h, a * h)
  go_ref[...] = dinv * jnp.dot(x2, w_ref[...],
                               preferred_element_type=jnp.float32)


def _post_body(p_ref, g_ref, dinv_ref, b_ref, batch_ref, lw_ref, lb_ref,
               out_ref):
  h = dinv_ref[...] * (p_ref[0:N] + p_ref[N_PAD:N_PAD + N] + g_ref[...]) \
      + b_ref[...]
  y = jnp.dot(h, lw_ref[...], preferred_element_type=jnp.float32)
  gid = lax.broadcasted_iota(jnp.int32, (G, 1), 0)
  oht = (gid == batch_ref[...]).astype(jnp.float32)
  seg = jnp.dot(oht, y, preferred_element_type=jnp.float32)
  cnt = jnp.sum(oht, axis=1, keepdims=True)
  logits = seg / jnp.maximum(cnt, 1.0) + lb_ref[...]
  m = jnp.max(logits, axis=1, keepdims=True)
  z = logits - m
  out_ref[...] = z - jnp.log(jnp.sum(jnp.exp(z), axis=1, keepdims=True))


_pre = pl.pallas_call(
    _pre_body,
    out_shape=(jax.ShapeDtypeStruct((N, D), jnp.float32),
               jax.ShapeDtypeStruct((N, 1), jnp.float32)))

_mid = pl.pallas_call(
    _mid_body, out_shape=jax.ShapeDtypeStruct((N, D), jnp.float32))

_post = pl.pallas_call(
    _post_body, out_shape=jax.ShapeDtypeStruct((G, OUT), jnp.float32))


def kernel(x, edge_index, batch, W0, b0, W1, b1, W2, b2, a0, a1, lin_W, lin_b):
  e = edge_index.shape[1]
  cpt = -(-e // (NW * CHUNK))
  cpt = -(-cpt // DBLK) * DBLK
  e_pad = NW * CHUNK * cpt
  pad = e_pad - e
  pid = jnp.arange(pad, dtype=jnp.int32)
  src = jnp.concatenate([edge_index[0], pid % N]).reshape(NW * cpt, CHUNK)
  dst = jnp.concatenate([edge_index[1], N + pid % (N_PAD - N)]
                        ).reshape(NW * cpt, CHUNK)

  deg_fn, scat_fn = _make_sc_kernels(cpt)

  degp = deg_fn(dst)
  dp = degp.reshape(NC, N_PAD)[:, :N].T

  g0, dinv = _pre(dp, x, W0)
  w_st = jnp.stack([W1, W2, W2])
  b_st = jnp.stack([b0.reshape(1, D), b1.reshape(1, D), b2.reshape(1, D)])
  a_st = jnp.stack([a0.reshape(1, 1), a1.reshape(1, 1), a1.reshape(1, 1)])
  pz = jnp.zeros((NC * N_PAD, D), jnp.float32)

  def layer(i, carry):
    g, _, _ = carry
    p = scat_fn(g, src, dst)
    go = _mid(p, g, dinv,
              lax.dynamic_index_in_dim(b_st, i, keepdims=False),
              lax.dynamic_index_in_dim(a_st, i, keepdims=False),
              lax.dynamic_index_in_dim(w_st, i, keepdims=False))
    return (go, g, p)

  trips = 3 + batch[0] // jnp.int32(2**31 - 1)
  _, g, p = lax.fori_loop(0, trips, layer, (g0, g0, pz))
  return _post(p, g, dinv, b2.reshape(1, D),
               batch.reshape(1, N), lin_W, lin_b.reshape(1, OUT))
